# baseline (speedup 1.0000x reference)
import functools

import jax
import jax.numpy as jnp
from jax import lax
from jax.experimental import pallas as pl
from jax.experimental.pallas import tpu as pltpu
from jax.experimental.pallas import tpu_sc as plsc

N_USER = 1000000
N_ITEM = 1000000
DIM = 64
BATCH = 16384

NC = 2
NS = 16
NW = NC * NS
B_PER_W = BATCH // NW
IDX_CHUNK = 128
N_CHUNK = B_PER_W // IDX_CHUNK
ROW_GROUPS = B_PER_W // 16


def _mf_body(u_hbm, i_hbm, U_hbm, V_hbm, out_hbm,
             u_idx, i_idx, u_rows, v_rows, out_v, sem):
    wid = lax.axis_index("s") * NC + lax.axis_index("c")
    base = wid * B_PER_W

    pltpu.sync_copy(u_hbm.at[pl.ds(wid * N_CHUNK, N_CHUNK)], u_idx)
    pltpu.sync_copy(i_hbm.at[pl.ds(wid * N_CHUNK, N_CHUNK)], i_idx)

    copies = []
    for j in range(N_CHUNK):
        copies.append(pltpu.make_async_copy(
            U_hbm.at[u_idx.at[j]],
            u_rows.at[pl.ds(j * IDX_CHUNK, IDX_CHUNK)], sem))
        copies.append(pltpu.make_async_copy(
            V_hbm.at[i_idx.at[j]],
            v_rows.at[pl.ds(j * IDX_CHUNK, IDX_CHUNK)], sem))
    for c in copies:
        c.start()
    for c in copies:
        c.wait()

    iota16 = lax.iota(jnp.int32, 16)

    def group(g, carry):
        rows = g * 16 + iota16
        acc = jnp.zeros((16,), jnp.float32)
        for d in range(DIM):
            cols = jnp.full((16,), d, jnp.int32)
            uu = plsc.load_gather(u_rows, [rows, cols])
            vv = plsc.load_gather(v_rows, [rows, cols])
            acc = acc + uu * vv
        out_v[pl.ds(g * 16, 16)] = acc
        return carry

    lax.fori_loop(0, ROW_GROUPS, group, 0, unroll=False)

    pltpu.sync_copy(out_v, out_hbm.at[pl.ds(base, B_PER_W)])


@jax.jit
def kernel(u, i, U_emb, V_emb):
    u2 = u.reshape(NW * N_CHUNK, IDX_CHUNK)
    i2 = i.reshape(NW * N_CHUNK, IDX_CHUNK)
    mesh = plsc.VectorSubcoreMesh(core_axis_name="c", subcore_axis_name="s")
    f = pl.kernel(
        _mf_body,
        out_type=jax.ShapeDtypeStruct((BATCH,), jnp.float32),
        mesh=mesh,
        compiler_params=pltpu.CompilerParams(
            needs_layout_passes=False, use_tc_tiling_on_sc=False),
        scratch_types=[
            pltpu.VMEM((N_CHUNK, IDX_CHUNK), jnp.int32),
            pltpu.VMEM((N_CHUNK, IDX_CHUNK), jnp.int32),
            pltpu.VMEM((B_PER_W, DIM), jnp.float32),
            pltpu.VMEM((B_PER_W, DIM), jnp.float32),
            pltpu.VMEM((B_PER_W,), jnp.float32),
            pltpu.SemaphoreType.DMA,
        ],
    )
    return f(u2, i2, U_emb, V_emb)
